# baseline (speedup 1.0000x reference)
import functools

import numpy as np
import jax
import jax.numpy as jnp
from jax import lax
from jax.experimental import pallas as pl
from jax.experimental.pallas import tpu as pltpu
from jax.experimental.pallas import tpu_sc as plsc

EMB = 32
NCOL = 9
NSEQ = 50
NB = 4096
N_TOK = NB * NSEQ
NW = 32
BPW = NB // NW
K = 64
NGRP = K // 16
SLAB = K * NSEQ * NCOL
PAD = 64
NTAB = 8
ROW = NTAB * EMB
WROWS = 154

_TABLES = [
    (0, 0.0, 7.0, 0.2),
    (1, 120.0, 180.0, 5.0),
    (2, 70.0, 180.0, 5.0),
    (3, 70.0, 150.0, 5.0),
    (4, 0.0, 95.0, 5.0),
    (5, 0.0, 40.0, 5.0),
    (7, 0.0, 2.0, 0.2),
    (8, 70.0, 150.0, 5.0),
]


def _make_binsx():
    flat = np.full((NTAB * PAD,), np.inf, dtype=np.float32)
    ns, offs = [], []
    row0 = 0
    for t, (_c, a, b, s) in enumerate(_TABLES):
        bins = np.asarray(np.arange(a, b + s * 0.5, s), dtype=np.float32)
        n = bins.shape[0]
        ns.append(n)
        offs.append(row0)
        row0 += n + 1
        flat[t * PAD] = np.nan
        flat[t * PAD + 1 : t * PAD + 1 + n] = bins
    return flat, ns, offs


_BINSX_NP, _NBINS, _OFFS = _make_binsx()


def _emb_body(par_hbm, binsx_hbm, wall_hbm, out_hbm,
              par_v0, par_v1, binsx_v, wall_v, dest_v0, dest_v1,
              ssem0, ssem1, wsem0, wsem1):
    pars = (par_v0, par_v1)
    dests = (dest_v0, dest_v1)
    ssems = (ssem0, ssem1)
    wsems = (wsem0, wsem1)
    wid = lax.axis_index("s") * 2 + lax.axis_index("c")
    b0w = wid * BPW

    pltpu.sync_copy(binsx_hbm, binsx_v)
    pltpu.sync_copy(wall_hbm, wall_v)
    for h in range(2):
        pltpu.async_copy(par_hbm.at[pl.ds((b0w + h * K) * NSEQ * NCOL, SLAB)],
                         pars[h], ssems[h])

    def _do_group(g, buf, par_s, vs9):
        lane = lax.iota(jnp.int32, 16) + g * 16
        vbase = lane * (NSEQ * NCOL) + vs9
        rows = []
        for t, (col, a, _stop, s) in enumerate(_TABLES):
            n = _NBINS[t]
            p = plsc.load_gather(par_s, [vbase + col])
            q = (p - np.float32(a)) * np.float32(1.0 / s)
            est = (q + np.float32(0.5)).astype(jnp.int32)
            k = jnp.clip(est, 0, n)
            lo = plsc.load_gather(binsx_v, [k + (t * PAD)])
            hi = plsc.load_gather(binsx_v, [k + (t * PAD + 1)])
            ids = k + jnp.where(p > hi, 1, 0) - jnp.where(p <= lo, 1, 0)
            ids = jnp.where(p != p, n, ids) + _OFFS[t]
            rows.append(ids * EMB)
        for i in range(16):
            tok = g * 16 + i
            for t in range(NTAB):
                rb = rows[t][i]
                v0 = wall_v[pl.ds(rb, 16)]
                v1 = wall_v[pl.ds(rb + 16, 16)]
                dests[buf][tok, pl.ds(t * EMB, 16)] = v0
                dests[buf][tok, pl.ds(t * EMB + 16, 16)] = v1

    def _chunk(s_idx, buf, par_s, base_row):
        @pl.when(s_idx >= 2)
        def _():
            pltpu.make_async_copy(dests[buf], out_hbm.at[pl.ds(0, K)],
                                  wsems[buf]).wait()
        vs9 = jnp.broadcast_to(s_idx * NCOL, (16,)).astype(jnp.int32)

        def _two(gg, cr):
            _do_group(2 * gg, buf, par_s, vs9)
            _do_group(2 * gg + 1, buf, par_s, vs9)
            return cr

        lax.fori_loop(0, NGRP // 2, _two, 0)
        pltpu.async_copy(dests[buf],
                         out_hbm.at[pl.ds(s_idx * NB + base_row, K)],
                         wsems[buf])

    for h in range(2):
        par_s = pars[h]
        base_row = b0w + h * K
        pltpu.make_async_copy(par_hbm.at[pl.ds(0, SLAB)], par_s,
                              ssems[h]).wait()

        def _pair(i, carry):
            _chunk(2 * i, 0, par_s, base_row)
            _chunk(2 * i + 1, 1, par_s, base_row)
            return carry

        lax.fori_loop(0, NSEQ // 2, _pair, 0)
        for buf in range(2):
            pltpu.make_async_copy(dests[buf], out_hbm.at[pl.ds(0, K)],
                                  wsems[buf]).wait()


_emb_kernel = functools.partial(
    pl.kernel,
    out_type=jax.ShapeDtypeStruct((N_TOK, ROW), jnp.float32),
    mesh=plsc.VectorSubcoreMesh(core_axis_name="c", subcore_axis_name="s"),
    compiler_params=pltpu.CompilerParams(use_tc_tiling_on_sc=True,
                                         needs_layout_passes=False),
    scratch_types=[
        pltpu.VMEM((SLAB,), jnp.float32),
        pltpu.VMEM((SLAB,), jnp.float32),
        pltpu.VMEM((NTAB * PAD,), jnp.float32),
        pltpu.VMEM((WROWS * EMB,), jnp.float32),
        pltpu.VMEM((K, ROW), jnp.float32),
        pltpu.VMEM((K, ROW), jnp.float32),
        pltpu.SemaphoreType.DMA,
        pltpu.SemaphoreType.DMA,
        pltpu.SemaphoreType.DMA,
        pltpu.SemaphoreType.DMA,
    ],
)(_emb_body)


def kernel(params, W0, W1, W2, W3, W4, W5, W6, W7):
    par = params.reshape(N_TOK * NCOL)
    binsx = jnp.asarray(_BINSX_NP)
    wall = jnp.concatenate([W0, W1, W2, W3, W4, W5, W6, W7], axis=0).reshape(-1)
    out = _emb_kernel(par, binsx, wall)
    return out.reshape(NSEQ, NB, ROW).transpose(1, 0, 2)


# speedup vs baseline: 1.2162x; 1.2162x over previous
import functools

import numpy as np
import jax
import jax.numpy as jnp
from jax import lax
from jax.experimental import pallas as pl
from jax.experimental.pallas import tpu as pltpu
from jax.experimental.pallas import tpu_sc as plsc

EMB = 32
NCOL = 9
NSEQ = 50
NB = 4096
N_TOK = NB * NSEQ
NW = 32
BPW = NB // NW
K = 128
NGRP = K // 16
SLAB = K * NSEQ * NCOL
PAD = 64
NTAB = 8
ROW = NTAB * EMB
WROWS = 154

_TABLES = [
    (0, 0.0, 7.0, 0.2),
    (1, 120.0, 180.0, 5.0),
    (2, 70.0, 180.0, 5.0),
    (3, 70.0, 150.0, 5.0),
    (4, 0.0, 95.0, 5.0),
    (5, 0.0, 40.0, 5.0),
    (7, 0.0, 2.0, 0.2),
    (8, 70.0, 150.0, 5.0),
]


def _make_binsx():
    flat = np.full((NTAB * PAD,), np.inf, dtype=np.float32)
    ns, offs = [], []
    row0 = 0
    for t, (_c, a, b, s) in enumerate(_TABLES):
        bins = np.asarray(np.arange(a, b + s * 0.5, s), dtype=np.float32)
        n = bins.shape[0]
        ns.append(n)
        offs.append(row0)
        row0 += n + 1
        flat[t * PAD] = np.nan
        flat[t * PAD + 1 : t * PAD + 1 + n] = bins
    return flat, ns, offs


_BINSX_NP, _NBINS, _OFFS = _make_binsx()


def _emb_body(par_hbm, binsx_hbm, wall_hbm, out_hbm,
              par_v0, binsx_v, wall_v, dest_v0, dest_v1,
              ssem0, ssem1, wsem0, wsem1):
    pars = (par_v0,)
    dests = (dest_v0, dest_v1)
    ssems = (ssem0, ssem1)
    wsems = (wsem0, wsem1)
    wid = lax.axis_index("s") * 2 + lax.axis_index("c")
    b0w = wid * BPW

    pltpu.sync_copy(binsx_hbm, binsx_v)
    pltpu.sync_copy(wall_hbm, wall_v)
    pltpu.async_copy(par_hbm.at[pl.ds(b0w * NSEQ * NCOL, SLAB)],
                     pars[0], ssems[0])

    def _do_group(g, buf, par_s, vs9):
        lane = lax.iota(jnp.int32, 16) + g * 16
        vbase = lane * (NSEQ * NCOL) + vs9
        rows = []
        for t, (col, a, _stop, s) in enumerate(_TABLES):
            n = _NBINS[t]
            p = plsc.load_gather(par_s, [vbase + col])
            q = (p - np.float32(a)) * np.float32(1.0 / s)
            est = (q + np.float32(0.5)).astype(jnp.int32)
            k = jnp.clip(est, 0, n)
            lo = plsc.load_gather(binsx_v, [k + (t * PAD)])
            hi = plsc.load_gather(binsx_v, [k + (t * PAD + 1)])
            ids = k + jnp.where(p > hi, 1, 0) - jnp.where(p <= lo, 1, 0)
            ids = jnp.where(p != p, n, ids) + _OFFS[t]
            rows.append(ids * EMB)
        for i in range(16):
            tok = g * 16 + i
            for t in range(NTAB):
                rb = rows[t][i]
                v0 = wall_v[pl.ds(rb, 16)]
                v1 = wall_v[pl.ds(rb + 16, 16)]
                dests[buf][tok, pl.ds(t * EMB, 16)] = v0
                dests[buf][tok, pl.ds(t * EMB + 16, 16)] = v1

    def _chunk(s_idx, buf, par_s, base_row):
        @pl.when(s_idx >= 2)
        def _():
            pltpu.make_async_copy(dests[buf], out_hbm.at[pl.ds(0, K)],
                                  wsems[buf]).wait()
        vs9 = jnp.broadcast_to(s_idx * NCOL, (16,)).astype(jnp.int32)
        lax.fori_loop(0, NGRP,
                      lambda g, cr: (_do_group(g, buf, par_s, vs9), cr)[1],
                      0)
        pltpu.async_copy(dests[buf],
                         out_hbm.at[pl.ds(s_idx * NB + base_row, K)],
                         wsems[buf])

    for h in range(1):
        par_s = pars[h]
        base_row = b0w + h * K
        pltpu.make_async_copy(par_hbm.at[pl.ds(0, SLAB)], par_s,
                              ssems[h]).wait()

        def _pair(i, carry):
            _chunk(2 * i, 0, par_s, base_row)
            _chunk(2 * i + 1, 1, par_s, base_row)
            return carry

        lax.fori_loop(0, NSEQ // 2, _pair, 0)
        for buf in range(2):
            pltpu.make_async_copy(dests[buf], out_hbm.at[pl.ds(0, K)],
                                  wsems[buf]).wait()


_emb_kernel = functools.partial(
    pl.kernel,
    out_type=jax.ShapeDtypeStruct((N_TOK, ROW), jnp.float32),
    mesh=plsc.VectorSubcoreMesh(core_axis_name="c", subcore_axis_name="s"),
    compiler_params=pltpu.CompilerParams(use_tc_tiling_on_sc=True,
                                         needs_layout_passes=False),
    scratch_types=[
        pltpu.VMEM((SLAB,), jnp.float32),
        pltpu.VMEM((NTAB * PAD,), jnp.float32),
        pltpu.VMEM((WROWS * EMB,), jnp.float32),
        pltpu.VMEM((K, ROW), jnp.float32),
        pltpu.VMEM((K, ROW), jnp.float32),
        pltpu.SemaphoreType.DMA,
        pltpu.SemaphoreType.DMA,
        pltpu.SemaphoreType.DMA,
        pltpu.SemaphoreType.DMA,
    ],
)(_emb_body)


def kernel(params, W0, W1, W2, W3, W4, W5, W6, W7):
    par = params.reshape(N_TOK * NCOL)
    binsx = jnp.asarray(_BINSX_NP)
    wall = jnp.concatenate([W0, W1, W2, W3, W4, W5, W6, W7], axis=0).reshape(-1)
    out = _emb_kernel(par, binsx, wall)
    return out.reshape(NSEQ, NB, ROW).transpose(1, 0, 2)
